# Initial kernel scaffold; baseline (speedup 1.0000x reference)
import jax
import jax.numpy as jnp
from jax.experimental import pallas as pl


def kernel(token_embeddings, rel_height, rel_width):
    raise NotImplementedError("write your pallas kernel here")


# speedup vs baseline: 16.6023x; 16.6023x over previous
import jax
import jax.numpy as jnp
from jax.experimental import pallas as pl
from jax.experimental.pallas import tpu as pltpu

NP = 256
H = 768
NB = 32
BI = 16
U_ROWS = 512


def _rpe_kernel(tok_ref, rh_ref, rw_ref, out_ref, u8_ref):
    pid = pl.program_id(0)

    @pl.when(pid == 0)
    def _build_tables():
        hw = jnp.concatenate([rh_ref[...], rw_ref[...]], axis=1)
        tok = tok_ref[...]
        cls_c = jax.lax.broadcasted_iota(jnp.int32, (U_ROWS, NB + 1), 1)
        cls_b = jax.lax.broadcasted_iota(jnp.int32, (U_ROWS, 2 * NB + 1), 1)
        m = jax.lax.broadcasted_iota(jnp.int32, (U_ROWS, 1), 0)
        for r in range(8):
            d = jnp.abs(255 - r - m)
            c = jnp.minimum(d, NB)
            b = jnp.minimum(d // 2, NB - 1)
            u = jnp.dot((c == cls_c).astype(jnp.float32), hw,
                        preferred_element_type=jnp.float32)
            u += jnp.dot((b == cls_b).astype(jnp.float32), tok,
                         preferred_element_type=jnp.float32)
            u8_ref[r] = u

    for ii in range(BI):
        s = 255 - (pid * BI + ii)
        r = jax.lax.rem(s, 8)
        q8 = pl.multiple_of(s - r, 8)
        out_ref[ii] = u8_ref[r, pl.ds(q8, NP), :]


@jax.jit
def kernel(token_embeddings, rel_height, rel_width):
    return pl.pallas_call(
        _rpe_kernel,
        grid=(NP // BI,),
        in_specs=[
            pl.BlockSpec((2 * NB + 1, H), lambda i: (0, 0)),
            pl.BlockSpec((NB + 1, H // 2), lambda i: (0, 0)),
            pl.BlockSpec((NB + 1, H // 2), lambda i: (0, 0)),
        ],
        out_specs=pl.BlockSpec((BI, NP, H), lambda i: (i, 0, 0)),
        out_shape=jax.ShapeDtypeStruct((NP, NP, H), jnp.float32),
        scratch_shapes=[pltpu.VMEM((8, U_ROWS, H), jnp.float32)],
    )(token_embeddings, rel_height, rel_width)
